# baseline (speedup 1.0000x reference)
import functools
import math

import jax
import jax.numpy as jnp
from jax import lax
from jax.experimental import pallas as pl
from jax.experimental.pallas import tpu as pltpu
from jax.experimental.pallas import tpu_sc as plsc

D_MODEL = 1024
B_TOTAL = 4 * 4096
NUM_CORES = 2
NUM_SUBCORES = 16
NW = NUM_CORES * NUM_SUBCORES
BPW = B_TOTAL // NW
CHUNK = 32
NCHUNK = BPW // CHUNK
LANES = 16
SCALE = math.sqrt(D_MODEL)

_mesh = plsc.VectorSubcoreMesh(core_axis_name="c", subcore_axis_name="s")


@functools.partial(
    pl.kernel,
    mesh=_mesh,
    out_type=jax.ShapeDtypeStruct((B_TOTAL, D_MODEL), jnp.float32),
    scratch_types=[
        pltpu.VMEM((BPW,), jnp.int32),
        pltpu.VMEM((CHUNK, D_MODEL), jnp.float32),
        pltpu.VMEM((CHUNK, D_MODEL), jnp.float32),
        pltpu.VMEM((CHUNK, D_MODEL), jnp.float32),
        pltpu.SemaphoreType.DMA,
        pltpu.SemaphoreType.DMA,
        pltpu.SemaphoreType.DMA,
        pltpu.SemaphoreType.DMA,
        pltpu.SemaphoreType.DMA,
        pltpu.SemaphoreType.DMA,
    ],
)
def _emb_lookup(idx_hbm, table_hbm, out_hbm, idx_v, rows0, rows1, rows2,
                g0, g1, g2, s0, s1, s2):
    wid = lax.axis_index("s") * NUM_CORES + lax.axis_index("c")
    base = wid * BPW
    bufs, gsems, ssems = (rows0, rows1, rows2), (g0, g1, g2), (s0, s1, s2)
    pltpu.sync_copy(idx_hbm.at[pl.ds(base, BPW)], idx_v)

    def gather(g, buf, sem):
        return pltpu.async_copy(
            table_hbm.at[idx_v.at[pl.ds(g * CHUNK, CHUNK)]], buf, sem
        )

    def scale(buf):
        def row_body(r, c2):
            for col in range(0, D_MODEL, LANES):
                buf[r, pl.ds(col, LANES)] = buf[r, pl.ds(col, LANES)] * SCALE
            return c2

        lax.fori_loop(0, CHUNK, row_body, 0)

    NBUF = 3
    pending = [None] * NBUF
    inflight = [None] * NBUF
    for g in range(min(2, NCHUNK)):
        inflight[g] = gather(g, bufs[g], gsems[g])
    for g in range(NCHUNK):
        b = g % NBUF
        nxt = g + 2
        if nxt < NCHUNK:
            nb = nxt % NBUF
            if pending[nb] is not None:
                pending[nb].wait()
            inflight[nb] = gather(nxt, bufs[nb], gsems[nb])
        inflight[b].wait()
        scale(bufs[b])
        pending[b] = pltpu.async_copy(
            bufs[b], out_hbm.at[pl.ds(base + g * CHUNK, CHUNK)], ssems[b]
        )
    for p in pending:
        if p is not None:
            p.wait()


def kernel(x, table):
    idx = jnp.reshape(x, (B_TOTAL,)).astype(jnp.int32)
    out = _emb_lookup(idx, table)
    return jnp.reshape(out, (*x.shape, D_MODEL))


# speedup vs baseline: 1.1233x; 1.1233x over previous
import functools
import math

import jax
import jax.numpy as jnp
from jax import lax
from jax.experimental import pallas as pl
from jax.experimental.pallas import tpu as pltpu
from jax.experimental.pallas import tpu_sc as plsc

D_MODEL = 1024
B_TOTAL = 4 * 4096
NUM_CORES = 2
NUM_SUBCORES = 16
NW = NUM_CORES * NUM_SUBCORES
BPW = B_TOTAL // NW
CHUNK = 32
NCHUNK = BPW // CHUNK
LANES = 16
SCALE = math.sqrt(D_MODEL)

_mesh = plsc.VectorSubcoreMesh(core_axis_name="c", subcore_axis_name="s")


@functools.partial(
    pl.kernel,
    mesh=_mesh,
    out_type=jax.ShapeDtypeStruct((B_TOTAL, D_MODEL), jnp.float32),
    scratch_types=[
        pltpu.VMEM((BPW,), jnp.int32),
        pltpu.VMEM((CHUNK, D_MODEL), jnp.float32),
        pltpu.VMEM((CHUNK, D_MODEL), jnp.float32),
        pltpu.VMEM((CHUNK, D_MODEL), jnp.float32),
        pltpu.SemaphoreType.DMA,
        pltpu.SemaphoreType.DMA,
        pltpu.SemaphoreType.DMA,
        pltpu.SemaphoreType.DMA,
        pltpu.SemaphoreType.DMA,
        pltpu.SemaphoreType.DMA,
    ],
)
def _emb_lookup(idx_hbm, table_hbm, out_hbm, idx_v, rows0, rows1, rows2,
                g0, g1, g2, s0, s1, s2):
    wid = lax.axis_index("s") * NUM_CORES + lax.axis_index("c")
    base = wid * BPW
    bufs, gsems, ssems = (rows0, rows1, rows2), (g0, g1, g2), (s0, s1, s2)
    pltpu.sync_copy(idx_hbm.at[pl.ds(base, BPW)], idx_v)

    def gather(g, buf, sem):
        return pltpu.async_copy(
            table_hbm.at[idx_v.at[pl.ds(g * CHUNK, CHUNK)]], buf, sem
        )

    def scale(buf):
        def row_body(r, c2):
            for col in range(0, D_MODEL, LANES):
                buf[r, pl.ds(col, LANES)] = buf[r, pl.ds(col, LANES)] * SCALE
            return c2

        lax.fori_loop(0, CHUNK, row_body, 0)

    NBUF = 3
    pending = [None] * NBUF
    inflight = [None] * NBUF
    inflight[0] = gather(0, bufs[0], gsems[0])
    for g in range(NCHUNK):
        b = g % NBUF
        nxt = g + 1
        if nxt < NCHUNK:
            nb = nxt % NBUF
            if pending[nb] is not None:
                pending[nb].wait()
            inflight[nb] = gather(nxt, bufs[nb], gsems[nb])
        inflight[b].wait()
        scale(bufs[b])
        pending[b] = pltpu.async_copy(
            bufs[b], out_hbm.at[pl.ds(base + g * CHUNK, CHUNK)], ssems[b]
        )
    for p in pending:
        if p is not None:
            p.wait()


def kernel(x, table):
    idx = jnp.reshape(x, (B_TOTAL,)).astype(jnp.int32)
    out = _emb_lookup(idx, table)
    return jnp.reshape(out, (*x.shape, D_MODEL))
